# baseline (speedup 1.0000x reference)
import functools
import math

import jax
import jax.numpy as jnp
from jax import lax
from jax.experimental import pallas as pl
from jax.experimental.pallas import tpu as pltpu
from jax.experimental.pallas import tpu_sc as plsc

D_MODEL = 64
SCALE = math.sqrt(D_MODEL)
LANES = 16

NC, NS = 2, 16
NW = NC * NS
BB = 128
NBUF = 5
AHEAD = 2


def _emb_kernel(batch: int, hist: int):
    nbb = batch // BB
    bb_per_w = nbb // NW
    steps = bb_per_w * hist
    fb = D_MODEL // 8

    mesh = plsc.VectorSubcoreMesh(core_axis_name="c", subcore_axis_name="s")

    @functools.partial(
        pl.kernel,
        out_type=jax.ShapeDtypeStruct((hist, fb, nbb, 8, BB), jnp.float32),
        mesh=mesh,
        compiler_params=pltpu.CompilerParams(
            use_tc_tiling_on_sc=False, needs_layout_passes=False
        ),
        scratch_types=dict(
            idx_t=pltpu.VMEM((hist, BB * bb_per_w), jnp.int32),
            gbufs=[pltpu.VMEM((BB, D_MODEL), jnp.float32) for _ in range(NBUF)],
            tbufs=[pltpu.VMEM((fb, 8, BB), jnp.float32) for _ in range(NBUF)],
            gsems=[pltpu.SemaphoreType.DMA for _ in range(NBUF)],
            ssems=[pltpu.SemaphoreType.DMA for _ in range(NBUF)],
        ),
    )
    def body(x_hbm, lut_hbm, out_hbm, idx_t, gbufs, tbufs, gsems, ssems):
        wid = lax.axis_index("s") * NC + lax.axis_index("c")

        pltpu.sync_copy(x_hbm.at[:, wid], idx_t)

        iot = lax.iota(jnp.int32, LANES)
        rowvecs = [iot + k * LANES for k in range(BB // LANES)]

        def start_gather(g, b):
            bl = g // hist
            h = g - bl * hist
            pltpu.async_copy(
                lut_hbm.at[idx_t.at[h, pl.ds(bl * BB, BB)]], gbufs[b], gsems[b]
            )

        def wait_gather(b):
            pltpu.make_async_copy(
                lut_hbm.at[idx_t.at[0, pl.ds(0, BB)]], gbufs[b], gsems[b]
            ).wait()

        def start_scatter(g, b):
            bl = g // hist
            h = g - bl * hist
            pltpu.async_copy(
                tbufs[b], out_hbm.at[h, :, wid * bb_per_w + bl], ssems[b]
            )

        def wait_scatter(b):
            pltpu.make_async_copy(
                tbufs[b], out_hbm.at[0, :, 0], ssems[b]
            ).wait()

        for g in range(AHEAD):
            start_gather(g, g)

        @pl.loop(0, steps, step=NBUF)
        def _steps(g0):
            for b in range(NBUF):
                g = g0 + b
                bn = (b + AHEAD) % NBUF

                @pl.when(g >= NBUF - AHEAD)
                def _():
                    wait_scatter(bn)

                @pl.when(g + AHEAD < steps)
                def _():
                    start_gather(g + AHEAD, bn)

                wait_gather(b)

                @pl.loop(0, D_MODEL, unroll=2)
                def _tr(f):
                    fvec = jnp.full((LANES,), f, jnp.int32)
                    for k in range(BB // LANES):
                        v = plsc.load_gather(gbufs[b], [rowvecs[k], fvec])
                        tbufs[b][f // 8, f % 8, pl.ds(k * LANES, LANES)] = v * SCALE

                start_scatter(g, b)

        for g in range(steps - (NBUF - AHEAD), steps):
            wait_scatter(g % NBUF)

    return body


def kernel(x, lut):
    batch, hist = x.shape
    xt3 = x.astype(jnp.int32).T.reshape(hist, NW, batch // NW)
    out5 = _emb_kernel(batch, hist)(xt3, lut)
    return out5.transpose(2, 4, 0, 1, 3).reshape(batch, hist, D_MODEL)


# speedup vs baseline: 1.2143x; 1.2143x over previous
import functools
import math

import jax
import jax.numpy as jnp
from jax import lax
from jax.experimental import pallas as pl
from jax.experimental.pallas import tpu as pltpu
from jax.experimental.pallas import tpu_sc as plsc

D_MODEL = 64
SCALE = math.sqrt(D_MODEL)
LANES = 16

NC, NS = 2, 16
NW = NC * NS
BB = 128
NBUF = 5
AHEAD = 2


def _emb_kernel(batch: int, hist: int):
    nbb = batch // BB
    bb_per_w = nbb // NW
    steps = bb_per_w * hist
    fb = D_MODEL // 8

    mesh = plsc.VectorSubcoreMesh(core_axis_name="c", subcore_axis_name="s")

    @functools.partial(
        pl.kernel,
        out_type=jax.ShapeDtypeStruct((hist, fb, nbb, 8, BB), jnp.float32),
        mesh=mesh,
        compiler_params=pltpu.CompilerParams(
            use_tc_tiling_on_sc=False, needs_layout_passes=False
        ),
        scratch_types=dict(
            idx_t=pltpu.VMEM((hist, BB * bb_per_w), jnp.int32),
            gbufs=[pltpu.VMEM((BB, D_MODEL), jnp.float32) for _ in range(NBUF)],
            tbufs=[pltpu.VMEM((fb, 8, BB), jnp.float32) for _ in range(NBUF)],
            gsems=[pltpu.SemaphoreType.DMA for _ in range(NBUF)],
            ssems=[pltpu.SemaphoreType.DMA for _ in range(NBUF)],
        ),
    )
    def body(x_hbm, lut_hbm, out_hbm, idx_t, gbufs, tbufs, gsems, ssems):
        wid = lax.axis_index("s") * NC + lax.axis_index("c")

        pltpu.sync_copy(x_hbm.at[:, wid], idx_t)

        iot = lax.iota(jnp.int32, LANES)
        rowvecs = [iot + k * LANES for k in range(BB // LANES)]

        def start_gather(g, b):
            bl = g // hist
            h = g - bl * hist
            pltpu.async_copy(
                lut_hbm.at[idx_t.at[h, pl.ds(bl * BB, BB)]], gbufs[b], gsems[b]
            )

        def wait_gather(b):
            pltpu.make_async_copy(
                lut_hbm.at[idx_t.at[0, pl.ds(0, BB)]], gbufs[b], gsems[b]
            ).wait()

        def start_scatter(g, b):
            bl = g // hist
            h = g - bl * hist
            pltpu.async_copy(
                tbufs[b], out_hbm.at[h, :, wid * bb_per_w + bl], ssems[b]
            )

        def wait_scatter(b):
            pltpu.make_async_copy(
                tbufs[b], out_hbm.at[0, :, 0], ssems[b]
            ).wait()

        for g in range(AHEAD):
            start_gather(g, g)

        @pl.loop(0, steps, step=NBUF)
        def _steps(g0):
            for b in range(NBUF):
                g = g0 + b
                bn = (b + AHEAD) % NBUF

                @pl.when(g >= NBUF - AHEAD)
                def _():
                    wait_scatter(bn)

                @pl.when(g + AHEAD < steps)
                def _():
                    start_gather(g + AHEAD, bn)

                wait_gather(b)

                @pl.loop(0, D_MODEL, unroll=2)
                def _tr(f):
                    fvec = jnp.full((LANES,), f, jnp.int32)
                    vals = [
                        plsc.load_gather(gbufs[b], [rowvecs[k], fvec])
                        for k in range(BB // LANES)
                    ]
                    for k in range(BB // LANES):
                        tbufs[b][f // 8, f % 8, pl.ds(k * LANES, LANES)] = (
                            vals[k] * SCALE
                        )

                start_scatter(g, b)

        for g in range(steps - (NBUF - AHEAD), steps):
            wait_scatter(g % NBUF)

    return body


def kernel(x, lut):
    batch, hist = x.shape
    xt3 = x.astype(jnp.int32).T.reshape(hist, NW, batch // NW)
    out5 = _emb_kernel(batch, hist)(xt3, lut)
    return out5.transpose(2, 4, 0, 1, 3).reshape(batch, hist, D_MODEL)


# speedup vs baseline: 1.7015x; 1.4012x over previous
import functools
import math

import jax
import jax.numpy as jnp
from jax import lax
from jax.experimental import pallas as pl
from jax.experimental.pallas import tpu as pltpu
from jax.experimental.pallas import tpu_sc as plsc

D_MODEL = 64
SCALE = math.sqrt(D_MODEL)
LANES = 16

NC, NS = 2, 16
NW = NC * NS
BB = 128
NBUF = 5
AHEAD = 2


def _emb_kernel(batch: int, hist: int):
    nbb = batch // BB
    bb_per_w = nbb // NW
    steps = bb_per_w * hist
    fb = D_MODEL // 8

    mesh = plsc.VectorSubcoreMesh(core_axis_name="c", subcore_axis_name="s")

    @functools.partial(
        pl.kernel,
        out_type=jax.ShapeDtypeStruct((hist, fb, nbb, 8, BB), jnp.float32),
        mesh=mesh,
        compiler_params=pltpu.CompilerParams(
            use_tc_tiling_on_sc=False, needs_layout_passes=False
        ),
        scratch_types=dict(
            idx_t=pltpu.VMEM((hist, BB * bb_per_w), jnp.int32),
            gbufs=[pltpu.VMEM((BB, D_MODEL), jnp.float32) for _ in range(NBUF)],
            tbufs=[pltpu.VMEM((fb, 8, BB), jnp.float32) for _ in range(NBUF)],
            gsems=[pltpu.SemaphoreType.DMA for _ in range(NBUF)],
            ssems=[pltpu.SemaphoreType.DMA for _ in range(NBUF)],
        ),
    )
    def body(x_hbm, lut_hbm, out_hbm, idx_t, gbufs, tbufs, gsems, ssems):
        wid = lax.axis_index("s") * NC + lax.axis_index("c")

        pltpu.sync_copy(x_hbm.at[:, wid], idx_t)

        iot = lax.iota(jnp.int32, LANES)
        rowvecs = [iot + k * LANES for k in range(BB // LANES)]

        def start_gather(g, b):
            bl = g // hist
            h = g - bl * hist
            pltpu.async_copy(
                lut_hbm.at[idx_t.at[h, pl.ds(bl * BB, BB)]], gbufs[b], gsems[b]
            )

        def wait_gather(b):
            pltpu.make_async_copy(
                lut_hbm.at[idx_t.at[0, pl.ds(0, BB)]], gbufs[b], gsems[b]
            ).wait()

        def start_scatter(g, b):
            bl = g // hist
            h = g - bl * hist
            pltpu.async_copy(
                tbufs[b], out_hbm.at[h, :, wid * bb_per_w + bl], ssems[b]
            )

        def wait_scatter(b):
            pltpu.make_async_copy(
                tbufs[b], out_hbm.at[0, :, 0], ssems[b]
            ).wait()

        for g in range(AHEAD):
            start_gather(g, g)

        @pl.loop(0, steps, step=NBUF)
        def _steps(g0):
            for b in range(NBUF):
                g = g0 + b
                bn = (b + AHEAD) % NBUF

                @pl.when(g >= NBUF - AHEAD)
                def _():
                    wait_scatter(bn)

                @pl.when(g + AHEAD < steps)
                def _():
                    start_gather(g + AHEAD, bn)

                wait_gather(b)

                @pl.loop(0, LANES)
                def _tr(j):
                    jd = (iot + j) & (LANES - 1)
                    for c0 in range(0, D_MODEL, LANES):
                        fvec = jd + c0
                        fbv = lax.shift_right_logical(fvec, 3)
                        fiv = fvec & 7
                        for k in range(BB // LANES):
                            v = plsc.load_gather(gbufs[b], [rowvecs[k], fvec])
                            plsc.store_scatter(
                                tbufs[b], [fbv, fiv, rowvecs[k]], v * SCALE
                            )

                start_scatter(g, b)

        for g in range(steps - (NBUF - AHEAD), steps):
            wait_scatter(g % NBUF)

    return body


def kernel(x, lut):
    batch, hist = x.shape
    xt3 = x.astype(jnp.int32).T.reshape(hist, NW, batch // NW)
    out5 = _emb_kernel(batch, hist)(xt3, lut)
    return out5.transpose(2, 4, 0, 1, 3).reshape(batch, hist, D_MODEL)


# speedup vs baseline: 2.5641x; 1.5070x over previous
import functools
import math

import jax
import jax.numpy as jnp
from jax import lax
from jax.experimental import pallas as pl
from jax.experimental.pallas import tpu as pltpu
from jax.experimental.pallas import tpu_sc as plsc

D_MODEL = 64
SCALE = math.sqrt(D_MODEL)
LANES = 16

NC, NS = 2, 16
NW = NC * NS
BB = 128
NBUF = 5
AHEAD = 2


def _emb_kernel(batch: int, hist: int):
    nbb = batch // BB
    bb_per_w = nbb // NW
    steps = bb_per_w * hist
    fb = D_MODEL // 8

    mesh = plsc.VectorSubcoreMesh(core_axis_name="c", subcore_axis_name="s")

    @functools.partial(
        pl.kernel,
        out_type=jax.ShapeDtypeStruct((hist, fb, nbb, 8, BB), jnp.float32),
        mesh=mesh,
        compiler_params=pltpu.CompilerParams(
            use_tc_tiling_on_sc=False, needs_layout_passes=False
        ),
        scratch_types=dict(
            idx_t=pltpu.VMEM((hist, BB * bb_per_w), jnp.int32),
            gbufs=[pltpu.VMEM((BB, D_MODEL), jnp.float32) for _ in range(NBUF)],
            tbufs=[pltpu.VMEM((fb, 8, BB), jnp.float32) for _ in range(NBUF)],
            gsems=[pltpu.SemaphoreType.DMA for _ in range(NBUF)],
            ssems=[pltpu.SemaphoreType.DMA for _ in range(NBUF)],
        ),
    )
    def body(x_hbm, lut_hbm, out_hbm, idx_t, gbufs, tbufs, gsems, ssems):
        wid = lax.axis_index("s") * NC + lax.axis_index("c")

        pltpu.sync_copy(x_hbm.at[:, wid], idx_t)

        iot = lax.iota(jnp.int32, LANES)
        rowvecs = [iot + k * LANES for k in range(BB // LANES)]

        def start_gather(g, b):
            bl = g // hist
            h = g - bl * hist
            pltpu.async_copy(
                lut_hbm.at[idx_t.at[h, pl.ds(bl * BB, BB)]], gbufs[b], gsems[b]
            )

        def wait_gather(b):
            pltpu.make_async_copy(
                lut_hbm.at[idx_t.at[0, pl.ds(0, BB)]], gbufs[b], gsems[b]
            ).wait()

        def start_scatter(g, b):
            bl = g // hist
            h = g - bl * hist
            pltpu.async_copy(
                tbufs[b], out_hbm.at[h, :, wid * bb_per_w + bl], ssems[b]
            )

        def wait_scatter(b):
            pltpu.make_async_copy(
                tbufs[b], out_hbm.at[0, :, 0], ssems[b]
            ).wait()

        for g in range(AHEAD):
            start_gather(g, g)

        @pl.loop(0, steps, step=NBUF)
        def _steps(g0):
            for b in range(NBUF):
                g = g0 + b
                bn = (b + AHEAD) % NBUF

                @pl.when(g >= NBUF - AHEAD)
                def _():
                    wait_scatter(bn)

                @pl.when(g + AHEAD < steps)
                def _():
                    start_gather(g + AHEAD, bn)

                wait_gather(b)

                @pl.loop(0, LANES)
                def _tr(j):
                    jd = (iot + j) & (LANES - 1)
                    cols = []
                    vals = []
                    for c0 in range(0, D_MODEL, LANES):
                        fvec = jd + c0
                        fbv = lax.shift_right_logical(fvec, 3)
                        fiv = fvec & 7
                        cols.append((fbv, fiv))
                        for k in range(BB // LANES):
                            vals.append(
                                plsc.load_gather(gbufs[b], [rowvecs[k], fvec])
                            )
                    for ci in range(D_MODEL // LANES):
                        fbv, fiv = cols[ci]
                        for k in range(BB // LANES):
                            plsc.store_scatter(
                                tbufs[b],
                                [fbv, fiv, rowvecs[k]],
                                vals[ci * (BB // LANES) + k] * SCALE,
                            )

                start_scatter(g, b)

        for g in range(steps - (NBUF - AHEAD), steps):
            wait_scatter(g % NBUF)

    return body


def kernel(x, lut):
    batch, hist = x.shape
    xt3 = x.astype(jnp.int32).T.reshape(hist, NW, batch // NW)
    out5 = _emb_kernel(batch, hist)(xt3, lut)
    return out5.transpose(2, 4, 0, 1, 3).reshape(batch, hist, D_MODEL)
